# baseline (speedup 1.0000x reference)
import functools

import jax
import jax.numpy as jnp
from jax import lax
from jax.experimental import pallas as pl
from jax.experimental.pallas import tpu as pltpu
from jax.experimental.pallas import tpu_sc as plsc

N_NODES = 10000
N_PAD = 10240
N_EDGES = 320000
NC = 2
NS = 16
NW = NC * NS
EPW = N_EDGES // NW
CHUNK = 128
NFULL = EPW // CHUNK
TAIL = EPW - NFULL * CHUNK
RPT = N_PAD // NS
RCH = 128
F32 = jnp.float32


def _sc_edge_kernel(D, with_cnt):
    gpr = D // 16

    def body(*refs):
        if with_cnt:
            (p_hbm, q_hbm, dst_hbm, src_hbm, s_out, cnt_out,
             dstv, srcv, dstv_t, srcv_t, a_v, b_v, a_t, b_t,
             ones_v, ones_t, s_sh, cnt_sh, sem1, sem2) = refs
        else:
            (p_hbm, q_hbm, dst_hbm, src_hbm, s_out,
             dstv, srcv, dstv_t, srcv_t, a_v, b_v, a_t, b_t,
             s_sh, sem1, sem2) = refs
        cid = lax.axis_index("c")
        sid = lax.axis_index("s")
        wid = sid * NC + cid
        zero16 = jnp.zeros((16,), F32)
        one16 = jnp.full((16,), 1.0, F32)

        def zrow(r, c):
            for g in range(gpr):
                a_v[r, pl.ds(g * 16, 16)] = zero16
            return c
        lax.fori_loop(0, CHUNK, zrow, 0)
        if with_cnt:
            for g in range(CHUNK // 16):
                ones_v[pl.ds(g * 16, 16)] = zero16
        for j in range(RPT // RCH):
            r0 = sid * RPT + j * RCH
            pltpu.sync_copy(a_v.at[pl.ds(0, RCH)], s_sh.at[pl.ds(r0, RCH)])
            if with_cnt:
                pltpu.sync_copy(ones_v.at[pl.ds(0, RCH)],
                                cnt_sh.at[pl.ds(r0, RCH)])
        if with_cnt:
            for g in range(CHUNK // 16):
                ones_v[pl.ds(g * 16, 16)] = one16
            ones_t[pl.ds(0, TAIL)] = one16
        plsc.subcore_barrier()

        ebase = wid * EPW

        def relu_add(av, bv, nrows):
            def row(r, c):
                for g in range(gpr):
                    sl = pl.ds(g * 16, 16)
                    av[r, sl] = jnp.maximum(av[r, sl] + bv[r, sl], 0.0)
                return c
            lax.fori_loop(0, nrows, row, 0, unroll=4)

        def chunk_body(c, carry):
            off = ebase + c * CHUNK
            ci1 = pltpu.async_copy(dst_hbm.at[pl.ds(off, CHUNK)], dstv, sem1)
            ci2 = pltpu.async_copy(src_hbm.at[pl.ds(off, CHUNK)], srcv, sem2)
            ci1.wait()
            ci2.wait()
            cp1 = pltpu.async_copy(p_hbm.at[dstv], a_v, sem1)
            cp2 = pltpu.async_copy(q_hbm.at[srcv], b_v, sem2)
            cp1.wait()
            cp2.wait()
            relu_add(a_v, b_v, CHUNK)
            pltpu.sync_copy(a_v, s_sh.at[dstv], add=True)
            if with_cnt:
                pltpu.sync_copy(ones_v, cnt_sh.at[dstv], add=True)
            return carry
        lax.fori_loop(0, NFULL, chunk_body, 0)

        off = ebase + NFULL * CHUNK
        pltpu.sync_copy(dst_hbm.at[pl.ds(off, TAIL)], dstv_t)
        pltpu.sync_copy(src_hbm.at[pl.ds(off, TAIL)], srcv_t)
        cp1 = pltpu.async_copy(p_hbm.at[dstv_t], a_t, sem1)
        cp2 = pltpu.async_copy(q_hbm.at[srcv_t], b_t, sem2)
        cp1.wait()
        cp2.wait()
        relu_add(a_t, b_t, TAIL)
        pltpu.sync_copy(a_t, s_sh.at[dstv_t], add=True)
        if with_cnt:
            pltpu.sync_copy(ones_t, cnt_sh.at[dstv_t], add=True)

        plsc.subcore_barrier()

        for j in range(RPT // RCH):
            r0 = sid * RPT + j * RCH
            pltpu.sync_copy(s_sh.at[pl.ds(r0, RCH)],
                            s_out.at[cid, pl.ds(r0, RCH)])
            if with_cnt:
                pltpu.sync_copy(cnt_sh.at[pl.ds(r0, RCH)],
                                cnt_out.at[cid, pl.ds(r0, RCH)])

    out_type = [jax.ShapeDtypeStruct((NC, N_PAD, D), F32)]
    scratch = [
        pltpu.VMEM((CHUNK,), jnp.int32),
        pltpu.VMEM((CHUNK,), jnp.int32),
        pltpu.VMEM((TAIL,), jnp.int32),
        pltpu.VMEM((TAIL,), jnp.int32),
        pltpu.VMEM((CHUNK, D), F32),
        pltpu.VMEM((CHUNK, D), F32),
        pltpu.VMEM((TAIL, D), F32),
        pltpu.VMEM((TAIL, D), F32),
    ]
    if with_cnt:
        out_type.append(jax.ShapeDtypeStruct((NC, N_PAD), F32))
        scratch += [
            pltpu.VMEM((CHUNK,), F32),
            pltpu.VMEM((TAIL,), F32),
        ]
    scratch.append(pltpu.VMEM_SHARED((N_PAD, D), F32))
    if with_cnt:
        scratch.append(pltpu.VMEM_SHARED((N_PAD,), F32))
    scratch += [pltpu.SemaphoreType.DMA, pltpu.SemaphoreType.DMA]

    mesh = plsc.VectorSubcoreMesh(core_axis_name="c", subcore_axis_name="s")
    return pl.kernel(body, out_type=out_type, mesh=mesh,
                     scratch_types=scratch, name=f"sc_edge_d{D}",
                     compiler_params=pltpu.CompilerParams(
                         use_tc_tiling_on_sc=False))


def _tc_proj1(x_ref, wdT_ref, wsT_ref, ba_ref, p_ref, q_ref):
    x = x_ref[...]
    p_ref[...] = jnp.dot(x, wdT_ref[...], preferred_element_type=F32) + ba_ref[...]
    q_ref[...] = jnp.dot(x, wsT_ref[...], preferred_element_type=F32)


def _tc_mid(sp_ref, cp_ref, wb1T_ref, bb1_ref, wd2T_ref, ws2T_ref, ba2_ref,
            p2_ref, q2_ref):
    s = sp_ref[0] + sp_ref[1]
    cnt = cp_ref[0] + cp_ref[1]
    inv = 1.0 / jnp.maximum(cnt, 1.0)
    ind = (cnt > 0.5).astype(F32)
    agg = jnp.dot(s * inv, wb1T_ref[...], preferred_element_type=F32)
    h1 = jnp.maximum(agg + bb1_ref[...] * ind, 0.0)
    p2_ref[...] = jnp.dot(h1, wd2T_ref[...], preferred_element_type=F32) + ba2_ref[...]
    q2_ref[...] = jnp.dot(h1, ws2T_ref[...], preferred_element_type=F32)


def _tc_head(sp2_ref, cp_ref, wb2T_ref, bb2_ref, state_ref,
             fc1T_ref, fc1b_ref, fc2T_ref, fc2b_ref,
             meanT_ref, meanb_ref, lsT_ref, lsb_ref,
             mean_ref, ls_ref):
    s = sp2_ref[0] + sp2_ref[1]
    cnt = cp_ref[0] + cp_ref[1]
    inv = 1.0 / jnp.maximum(cnt, 1.0)
    ind = (cnt > 0.5).astype(F32)
    agg = jnp.dot(s * inv, wb2T_ref[...], preferred_element_type=F32)
    agg = agg + bb2_ref[...] * ind
    ge = jnp.sum(agg, axis=0, keepdims=True) * (1.0 / N_NODES)
    z = jnp.concatenate([state_ref[...], ge], axis=1)
    z = jnp.maximum(jnp.dot(z, fc1T_ref[...], preferred_element_type=F32)
                    + fc1b_ref[...], 0.0)
    z = jnp.maximum(jnp.dot(z, fc2T_ref[...], preferred_element_type=F32)
                    + fc2b_ref[...], 0.0)
    mean_ref[...] = jnp.dot(z, meanT_ref[...], preferred_element_type=F32) + meanb_ref[...]
    ls_ref[...] = jnp.clip(
        jnp.dot(z, lsT_ref[...], preferred_element_type=F32) + lsb_ref[...],
        -20.0, 2.0)


def kernel(state, x, edge_index, g1_Wa, g1_ba, g1_Wb, g1_bb,
           g2_Wa, g2_ba, g2_Wb, g2_bb, fc1_W, fc1_b, fc2_W, fc2_b,
           mean_W, mean_b, ls_W, ls_b):
    S = x.shape[1]
    H = g1_Wb.shape[0]
    G = g2_Wb.shape[0]
    src = edge_index[0].astype(jnp.int32)
    dst = edge_index[1].astype(jnp.int32)

    p1, q1 = pl.pallas_call(
        _tc_proj1,
        out_shape=[jax.ShapeDtypeStruct((N_NODES, H), F32),
                   jax.ShapeDtypeStruct((N_NODES, H), F32)],
    )(x, g1_Wa[:, :S].T, g1_Wa[:, S:].T, g1_ba.reshape(1, H))

    sp1, cp = _sc_edge_kernel(H, True)(p1, q1, dst, src)
    cp3 = cp.reshape(NC, N_PAD, 1)

    p2, q2 = pl.pallas_call(
        _tc_mid,
        out_shape=[jax.ShapeDtypeStruct((N_PAD, G), F32),
                   jax.ShapeDtypeStruct((N_PAD, G), F32)],
    )(sp1, cp3, g1_Wb.T, g1_bb.reshape(1, H), g2_Wa[:, :H].T,
      g2_Wa[:, H:].T, g2_ba.reshape(1, G))

    (sp2,) = _sc_edge_kernel(G, False)(p2, q2, dst, src)

    mean, log_std = pl.pallas_call(
        _tc_head,
        out_shape=[jax.ShapeDtypeStruct((1, mean_W.shape[0]), F32),
                   jax.ShapeDtypeStruct((1, mean_W.shape[0]), F32)],
    )(sp2, cp3, g2_Wb.T, g2_bb.reshape(1, G), state,
      fc1_W.T, fc1_b.reshape(1, -1), fc2_W.T, fc2_b.reshape(1, -1),
      mean_W.T, mean_b.reshape(1, -1), ls_W.T, ls_b.reshape(1, -1))
    return (mean, log_std)


# speedup vs baseline: 1.5457x; 1.5457x over previous
import functools

import jax
import jax.numpy as jnp
from jax import lax
from jax.experimental import pallas as pl
from jax.experimental.pallas import tpu as pltpu
from jax.experimental.pallas import tpu_sc as plsc

N_NODES = 10000
N_PAD = 10240
N_EDGES = 320000
NC = 2
NS = 16
NW = NC * NS
EPW = N_EDGES // NW
CHUNK = 128
NFULL = EPW // CHUNK
TAIL = EPW - NFULL * CHUNK
RPT = N_PAD // NS
RCH = 128
F32 = jnp.float32


def _sc_edge_kernel(D, with_cnt):
    gpr = D // 16

    def body(*refs):
        if with_cnt:
            (p_hbm, q_hbm, dst_hbm, src_hbm, s_out, cnt_out,
             dstv, srcv, dstv_t, srcv_t, a_v, b_v, a_t, b_t,
             ones_v, ones_t, s_sh, cnt_sh, sem1, sem2) = refs
        else:
            (p_hbm, q_hbm, dst_hbm, src_hbm, s_out,
             dstv, srcv, dstv_t, srcv_t, a_v, b_v, a_t, b_t,
             s_sh, sem1, sem2) = refs
        cid = lax.axis_index("c")
        sid = lax.axis_index("s")
        wid = sid * NC + cid
        zero16 = jnp.zeros((16,), F32)
        one16 = jnp.full((16,), 1.0, F32)

        def zrow(r, c):
            for g in range(gpr):
                a_v[r, pl.ds(g * 16, 16)] = zero16
            return c
        lax.fori_loop(0, CHUNK, zrow, 0)
        if with_cnt:
            for g in range(CHUNK // 16):
                ones_v[pl.ds(g * 16, 16)] = zero16
        for j in range(RPT // RCH):
            r0 = sid * RPT + j * RCH
            pltpu.sync_copy(a_v.at[pl.ds(0, RCH)], s_sh.at[pl.ds(r0, RCH)])
            if with_cnt:
                pltpu.sync_copy(ones_v.at[pl.ds(0, RCH)],
                                cnt_sh.at[pl.ds(r0, RCH)])
        if with_cnt:
            for g in range(CHUNK // 16):
                ones_v[pl.ds(g * 16, 16)] = one16
            ones_t[pl.ds(0, TAIL)] = one16
        plsc.subcore_barrier()

        ebase = wid * EPW

        def relu_add(av, bv, nrows):
            def row(r, c):
                for g in range(gpr):
                    sl = pl.ds(g * 16, 16)
                    av[r, sl] = jnp.maximum(av[r, sl] + bv[r, sl], 0.0)
                return c
            lax.fori_loop(0, nrows, row, 0)

        def chunk_body(c, carry):
            off = ebase + c * CHUNK
            pltpu.sync_copy(dst_hbm.at[pl.ds(off, CHUNK)], dstv)
            pltpu.sync_copy(src_hbm.at[pl.ds(off, CHUNK)], srcv)
            cp1 = pltpu.async_copy(p_hbm.at[dstv], a_v, sem1)
            cp2 = pltpu.async_copy(q_hbm.at[srcv], b_v, sem2)
            cp1.wait()
            cp2.wait()
            relu_add(a_v, b_v, CHUNK)
            pltpu.sync_copy(a_v, s_sh.at[dstv], add=True)
            if with_cnt:
                pltpu.sync_copy(ones_v, cnt_sh.at[dstv], add=True)
            return carry
        lax.fori_loop(0, NFULL, chunk_body, 0)

        off = ebase + NFULL * CHUNK
        pltpu.sync_copy(dst_hbm.at[pl.ds(off, TAIL)], dstv_t)
        pltpu.sync_copy(src_hbm.at[pl.ds(off, TAIL)], srcv_t)
        cp1 = pltpu.async_copy(p_hbm.at[dstv_t], a_t, sem1)
        cp2 = pltpu.async_copy(q_hbm.at[srcv_t], b_t, sem2)
        cp1.wait()
        cp2.wait()
        relu_add(a_t, b_t, TAIL)
        pltpu.sync_copy(a_t, s_sh.at[dstv_t], add=True)
        if with_cnt:
            pltpu.sync_copy(ones_t, cnt_sh.at[dstv_t], add=True)

        plsc.subcore_barrier()

        for j in range(RPT // RCH):
            r0 = sid * RPT + j * RCH
            pltpu.sync_copy(s_sh.at[pl.ds(r0, RCH)],
                            s_out.at[cid, pl.ds(r0, RCH)])
            if with_cnt:
                pltpu.sync_copy(cnt_sh.at[pl.ds(r0, RCH)],
                                cnt_out.at[cid, pl.ds(r0, RCH)])

    out_type = [jax.ShapeDtypeStruct((NC, N_PAD, D), F32)]
    scratch = [
        pltpu.VMEM((CHUNK,), jnp.int32),
        pltpu.VMEM((CHUNK,), jnp.int32),
        pltpu.VMEM((TAIL,), jnp.int32),
        pltpu.VMEM((TAIL,), jnp.int32),
        pltpu.VMEM((CHUNK, D), F32),
        pltpu.VMEM((CHUNK, D), F32),
        pltpu.VMEM((TAIL, D), F32),
        pltpu.VMEM((TAIL, D), F32),
    ]
    if with_cnt:
        out_type.append(jax.ShapeDtypeStruct((NC, N_PAD), F32))
        scratch += [
            pltpu.VMEM((CHUNK,), F32),
            pltpu.VMEM((TAIL,), F32),
        ]
    scratch.append(pltpu.VMEM_SHARED((N_PAD, D), F32))
    if with_cnt:
        scratch.append(pltpu.VMEM_SHARED((N_PAD,), F32))
    scratch += [pltpu.SemaphoreType.DMA, pltpu.SemaphoreType.DMA]

    mesh = plsc.VectorSubcoreMesh(core_axis_name="c", subcore_axis_name="s")
    return pl.kernel(body, out_type=out_type, mesh=mesh,
                     scratch_types=scratch, name=f"sc_edge_d{D}",
                     compiler_params=pltpu.CompilerParams(
                         use_tc_tiling_on_sc=False))


def _tc_proj1(x_ref, wdT_ref, wsT_ref, ba_ref, p_ref, q_ref):
    x = x_ref[...]
    p_ref[...] = jnp.dot(x, wdT_ref[...], preferred_element_type=F32) + ba_ref[...]
    q_ref[...] = jnp.dot(x, wsT_ref[...], preferred_element_type=F32)


def _tc_mid(sp_ref, cp_ref, wb1T_ref, bb1_ref, wd2T_ref, ws2T_ref, ba2_ref,
            p2_ref, q2_ref):
    s = sp_ref[0] + sp_ref[1]
    cnt = cp_ref[0] + cp_ref[1]
    inv = 1.0 / jnp.maximum(cnt, 1.0)
    ind = (cnt > 0.5).astype(F32)
    agg = jnp.dot(s * inv, wb1T_ref[...], preferred_element_type=F32)
    h1 = jnp.maximum(agg + bb1_ref[...] * ind, 0.0)
    p2_ref[...] = jnp.dot(h1, wd2T_ref[...], preferred_element_type=F32) + ba2_ref[...]
    q2_ref[...] = jnp.dot(h1, ws2T_ref[...], preferred_element_type=F32)


def _tc_head(sp2_ref, cp_ref, wb2T_ref, bb2_ref, state_ref,
             fc1T_ref, fc1b_ref, fc2T_ref, fc2b_ref,
             meanT_ref, meanb_ref, lsT_ref, lsb_ref,
             mean_ref, ls_ref):
    s = sp2_ref[0] + sp2_ref[1]
    cnt = cp_ref[0] + cp_ref[1]
    inv = 1.0 / jnp.maximum(cnt, 1.0)
    ind = (cnt > 0.5).astype(F32)
    agg = jnp.dot(s * inv, wb2T_ref[...], preferred_element_type=F32)
    agg = agg + bb2_ref[...] * ind
    ge = jnp.sum(agg, axis=0, keepdims=True) * (1.0 / N_NODES)
    z = jnp.concatenate([state_ref[...], ge], axis=1)
    z = jnp.maximum(jnp.dot(z, fc1T_ref[...], preferred_element_type=F32)
                    + fc1b_ref[...], 0.0)
    z = jnp.maximum(jnp.dot(z, fc2T_ref[...], preferred_element_type=F32)
                    + fc2b_ref[...], 0.0)
    mean_ref[...] = jnp.dot(z, meanT_ref[...], preferred_element_type=F32) + meanb_ref[...]
    ls_ref[...] = jnp.clip(
        jnp.dot(z, lsT_ref[...], preferred_element_type=F32) + lsb_ref[...],
        -20.0, 2.0)


def kernel(state, x, edge_index, g1_Wa, g1_ba, g1_Wb, g1_bb,
           g2_Wa, g2_ba, g2_Wb, g2_bb, fc1_W, fc1_b, fc2_W, fc2_b,
           mean_W, mean_b, ls_W, ls_b):
    S = x.shape[1]
    H = g1_Wb.shape[0]
    G = g2_Wb.shape[0]
    src = edge_index[0].astype(jnp.int32)
    dst = edge_index[1].astype(jnp.int32)

    p1, q1 = pl.pallas_call(
        _tc_proj1,
        out_shape=[jax.ShapeDtypeStruct((N_NODES, H), F32),
                   jax.ShapeDtypeStruct((N_NODES, H), F32)],
    )(x, g1_Wa[:, :S].T, g1_Wa[:, S:].T, g1_ba.reshape(1, H))

    sp1, cp = _sc_edge_kernel(H, True)(p1, q1, dst, src)
    cp3 = cp.reshape(NC, N_PAD, 1)

    p2, q2 = pl.pallas_call(
        _tc_mid,
        out_shape=[jax.ShapeDtypeStruct((N_PAD, G), F32),
                   jax.ShapeDtypeStruct((N_PAD, G), F32)],
    )(sp1, cp3, g1_Wb.T, g1_bb.reshape(1, H), g2_Wa[:, :H].T,
      g2_Wa[:, H:].T, g2_ba.reshape(1, G))

    (sp2,) = _sc_edge_kernel(G, False)(p2, q2, dst, src)

    mean, log_std = pl.pallas_call(
        _tc_head,
        out_shape=[jax.ShapeDtypeStruct((1, mean_W.shape[0]), F32),
                   jax.ShapeDtypeStruct((1, mean_W.shape[0]), F32)],
    )(sp2, cp3, g2_Wb.T, g2_bb.reshape(1, G), state,
      fc1_W.T, fc1_b.reshape(1, -1), fc2_W.T, fc2_b.reshape(1, -1),
      mean_W.T, mean_b.reshape(1, -1), ls_W.T, ls_b.reshape(1, -1))
    return (mean, log_std)


# speedup vs baseline: 1.6990x; 1.0992x over previous
import functools

import jax
import jax.numpy as jnp
from jax import lax
from jax.experimental import pallas as pl
from jax.experimental.pallas import tpu as pltpu
from jax.experimental.pallas import tpu_sc as plsc

N_NODES = 10000
N_PAD = 10240
N_EDGES = 320000
NC = 2
NS = 16
NW = NC * NS
EPW = N_EDGES // NW
CHUNK = 128
NFULL = EPW // CHUNK
TAIL = EPW - NFULL * CHUNK
RPT = N_PAD // NS
RCH = 128
F32 = jnp.float32


def _sc_edge_kernel(D, with_cnt):
    gpr = D // 16

    def body(*refs):
        if with_cnt:
            (p_hbm, q_hbm, dst_hbm, src_hbm, s_out, cnt_out,
             dstv, srcv, dstv_t, srcv_t, a_v, b_v, a_t, b_t,
             ones_v, ones_t, s_sh, cnt_sh, sem1, sem2) = refs
        else:
            (p_hbm, q_hbm, dst_hbm, src_hbm, s_out,
             dstv, srcv, dstv_t, srcv_t, a_v, b_v, a_t, b_t,
             s_sh, sem1, sem2) = refs
        cid = lax.axis_index("c")
        sid = lax.axis_index("s")
        wid = sid * NC + cid
        zero16 = jnp.zeros((16,), F32)
        one16 = jnp.full((16,), 1.0, F32)

        def zrow(r, c):
            for g in range(gpr):
                a_v[r, pl.ds(g * 16, 16)] = zero16
            return c
        lax.fori_loop(0, CHUNK, zrow, 0)
        if with_cnt:
            for g in range(CHUNK // 16):
                ones_v[pl.ds(g * 16, 16)] = zero16
        for j in range(RPT // RCH):
            r0 = sid * RPT + j * RCH
            pltpu.sync_copy(a_v.at[pl.ds(0, RCH)], s_sh.at[pl.ds(r0, RCH)])
            if with_cnt:
                pltpu.sync_copy(ones_v.at[pl.ds(0, RCH)],
                                cnt_sh.at[pl.ds(r0, RCH)])
        if with_cnt:
            for g in range(CHUNK // 16):
                ones_v[pl.ds(g * 16, 16)] = one16
            ones_t[pl.ds(0, TAIL)] = one16
        plsc.subcore_barrier()

        ebase = wid * EPW

        def relu_add(av, bv, nrows):
            def row(r, c):
                for g in range(gpr):
                    sl = pl.ds(g * 16, 16)
                    av[r, sl] = jnp.maximum(av[r, sl] + bv[r, sl], 0.0)
                return c
            lax.fori_loop(0, nrows, row, 0)

        def chunk_body(c, carry):
            off = ebase + c * CHUNK
            ci1 = pltpu.async_copy(dst_hbm.at[pl.ds(off, CHUNK)], dstv, sem1)
            ci2 = pltpu.async_copy(src_hbm.at[pl.ds(off, CHUNK)], srcv, sem2)
            ci1.wait()
            ci2.wait()
            cp1 = pltpu.async_copy(p_hbm.at[dstv], a_v, sem1)
            cp2 = pltpu.async_copy(q_hbm.at[srcv], b_v, sem2)
            cp1.wait()
            cp2.wait()
            relu_add(a_v, b_v, CHUNK)
            pltpu.sync_copy(a_v, s_sh.at[dstv], add=True)
            if with_cnt:
                pltpu.sync_copy(ones_v, cnt_sh.at[dstv], add=True)
            return carry
        lax.fori_loop(0, NFULL, chunk_body, 0)

        off = ebase + NFULL * CHUNK
        pltpu.sync_copy(dst_hbm.at[pl.ds(off, TAIL)], dstv_t)
        pltpu.sync_copy(src_hbm.at[pl.ds(off, TAIL)], srcv_t)
        cp1 = pltpu.async_copy(p_hbm.at[dstv_t], a_t, sem1)
        cp2 = pltpu.async_copy(q_hbm.at[srcv_t], b_t, sem2)
        cp1.wait()
        cp2.wait()
        relu_add(a_t, b_t, TAIL)
        pltpu.sync_copy(a_t, s_sh.at[dstv_t], add=True)
        if with_cnt:
            pltpu.sync_copy(ones_t, cnt_sh.at[dstv_t], add=True)

        plsc.subcore_barrier()

        for j in range(RPT // RCH):
            r0 = sid * RPT + j * RCH
            pltpu.sync_copy(s_sh.at[pl.ds(r0, RCH)],
                            s_out.at[cid, pl.ds(r0, RCH)])
            if with_cnt:
                pltpu.sync_copy(cnt_sh.at[pl.ds(r0, RCH)],
                                cnt_out.at[cid, pl.ds(r0, RCH)])

    out_type = [jax.ShapeDtypeStruct((NC, N_PAD, D), F32)]
    scratch = [
        pltpu.VMEM((CHUNK,), jnp.int32),
        pltpu.VMEM((CHUNK,), jnp.int32),
        pltpu.VMEM((TAIL,), jnp.int32),
        pltpu.VMEM((TAIL,), jnp.int32),
        pltpu.VMEM((CHUNK, D), F32),
        pltpu.VMEM((CHUNK, D), F32),
        pltpu.VMEM((TAIL, D), F32),
        pltpu.VMEM((TAIL, D), F32),
    ]
    if with_cnt:
        out_type.append(jax.ShapeDtypeStruct((NC, N_PAD), F32))
        scratch += [
            pltpu.VMEM((CHUNK,), F32),
            pltpu.VMEM((TAIL,), F32),
        ]
    scratch.append(pltpu.VMEM_SHARED((N_PAD, D), F32))
    if with_cnt:
        scratch.append(pltpu.VMEM_SHARED((N_PAD,), F32))
    scratch += [pltpu.SemaphoreType.DMA, pltpu.SemaphoreType.DMA]

    mesh = plsc.VectorSubcoreMesh(core_axis_name="c", subcore_axis_name="s")
    return pl.kernel(body, out_type=out_type, mesh=mesh,
                     scratch_types=scratch, name=f"sc_edge_d{D}",
                     compiler_params=pltpu.CompilerParams(
                         use_tc_tiling_on_sc=False))


def _tc_proj1(x_ref, wdT_ref, wsT_ref, ba_ref, p_ref, q_ref):
    x = x_ref[...]
    p_ref[...] = jnp.dot(x, wdT_ref[...], preferred_element_type=F32) + ba_ref[...]
    q_ref[...] = jnp.dot(x, wsT_ref[...], preferred_element_type=F32)


def _tc_mid(sp_ref, cp_ref, wb1T_ref, bb1_ref, wd2T_ref, ws2T_ref, ba2_ref,
            p2_ref, q2_ref):
    s = sp_ref[0] + sp_ref[1]
    cnt = cp_ref[0] + cp_ref[1]
    inv = 1.0 / jnp.maximum(cnt, 1.0)
    ind = (cnt > 0.5).astype(F32)
    agg = jnp.dot(s * inv, wb1T_ref[...], preferred_element_type=F32)
    h1 = jnp.maximum(agg + bb1_ref[...] * ind, 0.0)
    p2_ref[...] = jnp.dot(h1, wd2T_ref[...], preferred_element_type=F32) + ba2_ref[...]
    q2_ref[...] = jnp.dot(h1, ws2T_ref[...], preferred_element_type=F32)


def _tc_head(sp2_ref, cp_ref, wb2T_ref, bb2_ref, state_ref,
             fc1T_ref, fc1b_ref, fc2T_ref, fc2b_ref,
             meanT_ref, meanb_ref, lsT_ref, lsb_ref,
             mean_ref, ls_ref):
    s = sp2_ref[0] + sp2_ref[1]
    cnt = cp_ref[0] + cp_ref[1]
    inv = 1.0 / jnp.maximum(cnt, 1.0)
    ind = (cnt > 0.5).astype(F32)
    agg = jnp.dot(s * inv, wb2T_ref[...], preferred_element_type=F32)
    agg = agg + bb2_ref[...] * ind
    ge = jnp.sum(agg, axis=0, keepdims=True) * (1.0 / N_NODES)
    z = jnp.concatenate([state_ref[...], ge], axis=1)
    z = jnp.maximum(jnp.dot(z, fc1T_ref[...], preferred_element_type=F32)
                    + fc1b_ref[...], 0.0)
    z = jnp.maximum(jnp.dot(z, fc2T_ref[...], preferred_element_type=F32)
                    + fc2b_ref[...], 0.0)
    mean_ref[...] = jnp.dot(z, meanT_ref[...], preferred_element_type=F32) + meanb_ref[...]
    ls_ref[...] = jnp.clip(
        jnp.dot(z, lsT_ref[...], preferred_element_type=F32) + lsb_ref[...],
        -20.0, 2.0)


def kernel(state, x, edge_index, g1_Wa, g1_ba, g1_Wb, g1_bb,
           g2_Wa, g2_ba, g2_Wb, g2_bb, fc1_W, fc1_b, fc2_W, fc2_b,
           mean_W, mean_b, ls_W, ls_b):
    S = x.shape[1]
    H = g1_Wb.shape[0]
    G = g2_Wb.shape[0]
    src = edge_index[0].astype(jnp.int32)
    dst = edge_index[1].astype(jnp.int32)

    p1, q1 = pl.pallas_call(
        _tc_proj1,
        out_shape=[jax.ShapeDtypeStruct((N_NODES, H), F32),
                   jax.ShapeDtypeStruct((N_NODES, H), F32)],
    )(x, g1_Wa[:, :S].T, g1_Wa[:, S:].T, g1_ba.reshape(1, H))

    sp1, cp = _sc_edge_kernel(H, True)(p1, q1, dst, src)
    cp3 = cp.reshape(NC, N_PAD, 1)

    p2, q2 = pl.pallas_call(
        _tc_mid,
        out_shape=[jax.ShapeDtypeStruct((N_PAD, G), F32),
                   jax.ShapeDtypeStruct((N_PAD, G), F32)],
    )(sp1, cp3, g1_Wb.T, g1_bb.reshape(1, H), g2_Wa[:, :H].T,
      g2_Wa[:, H:].T, g2_ba.reshape(1, G))

    (sp2,) = _sc_edge_kernel(G, False)(p2, q2, dst, src)

    mean, log_std = pl.pallas_call(
        _tc_head,
        out_shape=[jax.ShapeDtypeStruct((1, mean_W.shape[0]), F32),
                   jax.ShapeDtypeStruct((1, mean_W.shape[0]), F32)],
    )(sp2, cp3, g2_Wb.T, g2_bb.reshape(1, G), state,
      fc1_W.T, fc1_b.reshape(1, -1), fc2_W.T, fc2_b.reshape(1, -1),
      mean_W.T, mean_b.reshape(1, -1), ls_W.T, ls_b.reshape(1, -1))
    return (mean, log_std)


# speedup vs baseline: 1.7116x; 1.0074x over previous
import functools

import jax
import jax.numpy as jnp
from jax import lax
from jax.experimental import pallas as pl
from jax.experimental.pallas import tpu as pltpu
from jax.experimental.pallas import tpu_sc as plsc

N_NODES = 10000
N_PAD = 10240
N_EDGES = 320000
NC = 2
NS = 16
NW = NC * NS
EPW = N_EDGES // NW
CHUNK = 128
NFULL = EPW // CHUNK
TAIL = EPW - NFULL * CHUNK
RPT = N_PAD // NS
RCH = 128
F32 = jnp.float32


def _sc_edge_kernel(D, with_cnt):
    gpr = D // 16

    def body(*refs):
        if with_cnt:
            (p_hbm, q_hbm, dst_hbm, src_hbm, s_out, cnt_out,
             dstv, srcv, dstv_t, srcv_t, a_v, b_v, a_t, b_t,
             ones_v, ones_t, s_sh, cnt_sh, sem1, sem2) = refs
        else:
            (p_hbm, q_hbm, dst_hbm, src_hbm, s_out,
             dstv, srcv, dstv_t, srcv_t, a_v, b_v, a_t, b_t,
             s_sh, sem1, sem2) = refs
        cid = lax.axis_index("c")
        sid = lax.axis_index("s")
        wid = sid * NC + cid
        zero16 = jnp.zeros((16,), F32)
        one16 = jnp.full((16,), 1.0, F32)

        def zrow(r, c):
            for g in range(gpr):
                a_v[r, pl.ds(g * 16, 16)] = zero16
            return c
        lax.fori_loop(0, CHUNK, zrow, 0)
        if with_cnt:
            for g in range(CHUNK // 16):
                ones_v[pl.ds(g * 16, 16)] = zero16
        for j in range(RPT // RCH):
            r0 = sid * RPT + j * RCH
            pltpu.sync_copy(a_v.at[pl.ds(0, RCH)], s_sh.at[pl.ds(r0, RCH)])
            if with_cnt:
                pltpu.sync_copy(ones_v.at[pl.ds(0, RCH)],
                                cnt_sh.at[pl.ds(r0, RCH)])
        if with_cnt:
            for g in range(CHUNK // 16):
                ones_v[pl.ds(g * 16, 16)] = one16
            ones_t[pl.ds(0, TAIL)] = one16
        plsc.subcore_barrier()

        ebase = wid * EPW

        def relu_add(av, bv, nrows):
            def row(r, c):
                for g in range(gpr):
                    sl = pl.ds(g * 16, 16)
                    av[r, sl] = jnp.maximum(av[r, sl] + bv[r, sl], 0.0)
                return c
            lax.fori_loop(0, nrows, row, 0)

        def chunk_body(c, carry):
            off = ebase + c * CHUNK
            ci1 = pltpu.async_copy(dst_hbm.at[pl.ds(off, CHUNK)], dstv, sem1)
            ci2 = pltpu.async_copy(src_hbm.at[pl.ds(off, CHUNK)], srcv, sem2)
            ci1.wait()
            ci2.wait()
            cp1 = pltpu.async_copy(p_hbm.at[dstv], a_v, sem1)
            cp2 = pltpu.async_copy(q_hbm.at[srcv], b_v, sem2)
            cp1.wait()
            cp2.wait()
            relu_add(a_v, b_v, CHUNK)
            if with_cnt:
                cs1 = pltpu.async_copy(a_v, s_sh.at[dstv], sem1, add=True)
                cs2 = pltpu.async_copy(ones_v, cnt_sh.at[dstv], sem2,
                                       add=True)
                cs1.wait()
                cs2.wait()
            else:
                pltpu.sync_copy(a_v, s_sh.at[dstv], add=True)
            return carry
        lax.fori_loop(0, NFULL, chunk_body, 0)

        off = ebase + NFULL * CHUNK
        pltpu.sync_copy(dst_hbm.at[pl.ds(off, TAIL)], dstv_t)
        pltpu.sync_copy(src_hbm.at[pl.ds(off, TAIL)], srcv_t)
        cp1 = pltpu.async_copy(p_hbm.at[dstv_t], a_t, sem1)
        cp2 = pltpu.async_copy(q_hbm.at[srcv_t], b_t, sem2)
        cp1.wait()
        cp2.wait()
        relu_add(a_t, b_t, TAIL)
        pltpu.sync_copy(a_t, s_sh.at[dstv_t], add=True)
        if with_cnt:
            pltpu.sync_copy(ones_t, cnt_sh.at[dstv_t], add=True)

        plsc.subcore_barrier()

        for j in range(RPT // RCH):
            r0 = sid * RPT + j * RCH
            pltpu.sync_copy(s_sh.at[pl.ds(r0, RCH)],
                            s_out.at[cid, pl.ds(r0, RCH)])
            if with_cnt:
                pltpu.sync_copy(cnt_sh.at[pl.ds(r0, RCH)],
                                cnt_out.at[cid, pl.ds(r0, RCH)])

    out_type = [jax.ShapeDtypeStruct((NC, N_PAD, D), F32)]
    scratch = [
        pltpu.VMEM((CHUNK,), jnp.int32),
        pltpu.VMEM((CHUNK,), jnp.int32),
        pltpu.VMEM((TAIL,), jnp.int32),
        pltpu.VMEM((TAIL,), jnp.int32),
        pltpu.VMEM((CHUNK, D), F32),
        pltpu.VMEM((CHUNK, D), F32),
        pltpu.VMEM((TAIL, D), F32),
        pltpu.VMEM((TAIL, D), F32),
    ]
    if with_cnt:
        out_type.append(jax.ShapeDtypeStruct((NC, N_PAD), F32))
        scratch += [
            pltpu.VMEM((CHUNK,), F32),
            pltpu.VMEM((TAIL,), F32),
        ]
    scratch.append(pltpu.VMEM_SHARED((N_PAD, D), F32))
    if with_cnt:
        scratch.append(pltpu.VMEM_SHARED((N_PAD,), F32))
    scratch += [pltpu.SemaphoreType.DMA, pltpu.SemaphoreType.DMA]

    mesh = plsc.VectorSubcoreMesh(core_axis_name="c", subcore_axis_name="s")
    return pl.kernel(body, out_type=out_type, mesh=mesh,
                     scratch_types=scratch, name=f"sc_edge_d{D}",
                     compiler_params=pltpu.CompilerParams(
                         use_tc_tiling_on_sc=False))


def _tc_proj1(x_ref, wdT_ref, wsT_ref, ba_ref, p_ref, q_ref):
    x = x_ref[...]
    p_ref[...] = jnp.dot(x, wdT_ref[...], preferred_element_type=F32) + ba_ref[...]
    q_ref[...] = jnp.dot(x, wsT_ref[...], preferred_element_type=F32)


def _tc_mid(sp_ref, cp_ref, wb1T_ref, bb1_ref, wd2T_ref, ws2T_ref, ba2_ref,
            p2_ref, q2_ref):
    s = sp_ref[0] + sp_ref[1]
    cnt = cp_ref[0] + cp_ref[1]
    inv = 1.0 / jnp.maximum(cnt, 1.0)
    ind = (cnt > 0.5).astype(F32)
    agg = jnp.dot(s * inv, wb1T_ref[...], preferred_element_type=F32)
    h1 = jnp.maximum(agg + bb1_ref[...] * ind, 0.0)
    p2_ref[...] = jnp.dot(h1, wd2T_ref[...], preferred_element_type=F32) + ba2_ref[...]
    q2_ref[...] = jnp.dot(h1, ws2T_ref[...], preferred_element_type=F32)


def _tc_head(sp2_ref, cp_ref, wb2T_ref, bb2_ref, state_ref,
             fc1T_ref, fc1b_ref, fc2T_ref, fc2b_ref,
             meanT_ref, meanb_ref, lsT_ref, lsb_ref,
             mean_ref, ls_ref):
    s = sp2_ref[0] + sp2_ref[1]
    cnt = cp_ref[0] + cp_ref[1]
    inv = 1.0 / jnp.maximum(cnt, 1.0)
    ind = (cnt > 0.5).astype(F32)
    agg = jnp.dot(s * inv, wb2T_ref[...], preferred_element_type=F32)
    agg = agg + bb2_ref[...] * ind
    ge = jnp.sum(agg, axis=0, keepdims=True) * (1.0 / N_NODES)
    z = jnp.concatenate([state_ref[...], ge], axis=1)
    z = jnp.maximum(jnp.dot(z, fc1T_ref[...], preferred_element_type=F32)
                    + fc1b_ref[...], 0.0)
    z = jnp.maximum(jnp.dot(z, fc2T_ref[...], preferred_element_type=F32)
                    + fc2b_ref[...], 0.0)
    mean_ref[...] = jnp.dot(z, meanT_ref[...], preferred_element_type=F32) + meanb_ref[...]
    ls_ref[...] = jnp.clip(
        jnp.dot(z, lsT_ref[...], preferred_element_type=F32) + lsb_ref[...],
        -20.0, 2.0)


def kernel(state, x, edge_index, g1_Wa, g1_ba, g1_Wb, g1_bb,
           g2_Wa, g2_ba, g2_Wb, g2_bb, fc1_W, fc1_b, fc2_W, fc2_b,
           mean_W, mean_b, ls_W, ls_b):
    S = x.shape[1]
    H = g1_Wb.shape[0]
    G = g2_Wb.shape[0]
    src = edge_index[0].astype(jnp.int32)
    dst = edge_index[1].astype(jnp.int32)

    p1, q1 = pl.pallas_call(
        _tc_proj1,
        out_shape=[jax.ShapeDtypeStruct((N_NODES, H), F32),
                   jax.ShapeDtypeStruct((N_NODES, H), F32)],
    )(x, g1_Wa[:, :S].T, g1_Wa[:, S:].T, g1_ba.reshape(1, H))

    sp1, cp = _sc_edge_kernel(H, True)(p1, q1, dst, src)
    cp3 = cp.reshape(NC, N_PAD, 1)

    p2, q2 = pl.pallas_call(
        _tc_mid,
        out_shape=[jax.ShapeDtypeStruct((N_PAD, G), F32),
                   jax.ShapeDtypeStruct((N_PAD, G), F32)],
    )(sp1, cp3, g1_Wb.T, g1_bb.reshape(1, H), g2_Wa[:, :H].T,
      g2_Wa[:, H:].T, g2_ba.reshape(1, G))

    (sp2,) = _sc_edge_kernel(G, False)(p2, q2, dst, src)

    mean, log_std = pl.pallas_call(
        _tc_head,
        out_shape=[jax.ShapeDtypeStruct((1, mean_W.shape[0]), F32),
                   jax.ShapeDtypeStruct((1, mean_W.shape[0]), F32)],
    )(sp2, cp3, g2_Wb.T, g2_bb.reshape(1, G), state,
      fc1_W.T, fc1_b.reshape(1, -1), fc2_W.T, fc2_b.reshape(1, -1),
      mean_W.T, mean_b.reshape(1, -1), ls_W.T, ls_b.reshape(1, -1))
    return (mean, log_std)


# speedup vs baseline: 1.8045x; 1.0542x over previous
import functools

import jax
import jax.numpy as jnp
from jax import lax
from jax.experimental import pallas as pl
from jax.experimental.pallas import tpu as pltpu
from jax.experimental.pallas import tpu_sc as plsc

N_NODES = 10000
N_PAD = 10240
N_EDGES = 320000
NC = 2
NS = 16
NW = NC * NS
EPW = N_EDGES // NW
CHUNK = 128
NFULL = EPW // CHUNK
TAIL = EPW - NFULL * CHUNK
RPT = N_PAD // NS
RCH = 128
F32 = jnp.float32


def _sc_edge_kernel(D, with_cnt):
    gpr = D // 16

    def body(*refs):
        if with_cnt:
            (p_hbm, q_hbm, dst_hbm, src_hbm, s_out, cnt_out,
             dstv, srcv, dstv1, srcv1, dstv_t, srcv_t, a_v, b_v, a_t, b_t,
             ones_v, ones_t, s_sh, cnt_sh, sem1, sem2, sem3, sem4) = refs
        else:
            (p_hbm, q_hbm, dst_hbm, src_hbm, s_out,
             dstv, srcv, dstv1, srcv1, dstv_t, srcv_t, a_v, b_v, a_t, b_t,
             s_sh, sem1, sem2, sem3, sem4) = refs
        cid = lax.axis_index("c")
        sid = lax.axis_index("s")
        wid = sid * NC + cid
        zero16 = jnp.zeros((16,), F32)
        one16 = jnp.full((16,), 1.0, F32)

        def zrow(r, c):
            for g in range(gpr):
                a_v[r, pl.ds(g * 16, 16)] = zero16
            return c
        lax.fori_loop(0, CHUNK, zrow, 0)
        if with_cnt:
            for g in range(CHUNK // 16):
                ones_v[pl.ds(g * 16, 16)] = zero16
        for j in range(RPT // RCH):
            r0 = sid * RPT + j * RCH
            pltpu.sync_copy(a_v.at[pl.ds(0, RCH)], s_sh.at[pl.ds(r0, RCH)])
            if with_cnt:
                pltpu.sync_copy(ones_v.at[pl.ds(0, RCH)],
                                cnt_sh.at[pl.ds(r0, RCH)])
        if with_cnt:
            for g in range(CHUNK // 16):
                ones_v[pl.ds(g * 16, 16)] = one16
            ones_t[pl.ds(0, TAIL)] = one16
        plsc.subcore_barrier()

        ebase = wid * EPW

        def relu_add(av, bv, nrows):
            def row(r, c):
                for g in range(gpr):
                    sl = pl.ds(g * 16, 16)
                    av[r, sl] = jnp.maximum(av[r, sl] + bv[r, sl], 0.0)
                return c
            lax.fori_loop(0, nrows, row, 0)

        def pair_body(t, carry):
            off0 = ebase + (2 * t) * CHUNK
            off1 = off0 + CHUNK
            i00 = pltpu.async_copy(dst_hbm.at[pl.ds(off0, CHUNK)], dstv,
                                   sem1)
            i01 = pltpu.async_copy(src_hbm.at[pl.ds(off0, CHUNK)], srcv,
                                   sem2)
            i10 = pltpu.async_copy(dst_hbm.at[pl.ds(off1, CHUNK)], dstv1,
                                   sem3)
            i11 = pltpu.async_copy(src_hbm.at[pl.ds(off1, CHUNK)], srcv1,
                                   sem4)

            def half(iw0, iw1, dv, sv):
                iw0.wait()
                iw1.wait()
                cp1 = pltpu.async_copy(p_hbm.at[dv], a_v, sem1)
                cp2 = pltpu.async_copy(q_hbm.at[sv], b_v, sem2)
                cp1.wait()
                cp2.wait()
                relu_add(a_v, b_v, CHUNK)
                if with_cnt:
                    cs1 = pltpu.async_copy(a_v, s_sh.at[dv], sem1,
                                           add=True)
                    cs2 = pltpu.async_copy(ones_v, cnt_sh.at[dv], sem2,
                                           add=True)
                    cs1.wait()
                    cs2.wait()
                else:
                    pltpu.sync_copy(a_v, s_sh.at[dv], add=True)
            half(i00, i01, dstv, srcv)
            half(i10, i11, dstv1, srcv1)
            return carry
        lax.fori_loop(0, NFULL // 2, pair_body, 0)

        off = ebase + NFULL * CHUNK
        pltpu.sync_copy(dst_hbm.at[pl.ds(off, TAIL)], dstv_t)
        pltpu.sync_copy(src_hbm.at[pl.ds(off, TAIL)], srcv_t)
        cp1 = pltpu.async_copy(p_hbm.at[dstv_t], a_t, sem1)
        cp2 = pltpu.async_copy(q_hbm.at[srcv_t], b_t, sem2)
        cp1.wait()
        cp2.wait()
        relu_add(a_t, b_t, TAIL)
        pltpu.sync_copy(a_t, s_sh.at[dstv_t], add=True)
        if with_cnt:
            pltpu.sync_copy(ones_t, cnt_sh.at[dstv_t], add=True)

        plsc.subcore_barrier()

        for j in range(RPT // RCH):
            r0 = sid * RPT + j * RCH
            pltpu.sync_copy(s_sh.at[pl.ds(r0, RCH)],
                            s_out.at[cid, pl.ds(r0, RCH)])
            if with_cnt:
                pltpu.sync_copy(cnt_sh.at[pl.ds(r0, RCH)],
                                cnt_out.at[cid, pl.ds(r0, RCH)])

    out_type = [jax.ShapeDtypeStruct((NC, N_PAD, D), F32)]
    scratch = [
        pltpu.VMEM((CHUNK,), jnp.int32),
        pltpu.VMEM((CHUNK,), jnp.int32),
        pltpu.VMEM((CHUNK,), jnp.int32),
        pltpu.VMEM((CHUNK,), jnp.int32),
        pltpu.VMEM((TAIL,), jnp.int32),
        pltpu.VMEM((TAIL,), jnp.int32),
        pltpu.VMEM((CHUNK, D), F32),
        pltpu.VMEM((CHUNK, D), F32),
        pltpu.VMEM((TAIL, D), F32),
        pltpu.VMEM((TAIL, D), F32),
    ]
    if with_cnt:
        out_type.append(jax.ShapeDtypeStruct((NC, N_PAD), F32))
        scratch += [
            pltpu.VMEM((CHUNK,), F32),
            pltpu.VMEM((TAIL,), F32),
        ]
    scratch.append(pltpu.VMEM_SHARED((N_PAD, D), F32))
    if with_cnt:
        scratch.append(pltpu.VMEM_SHARED((N_PAD,), F32))
    scratch += [pltpu.SemaphoreType.DMA for _ in range(4)]

    mesh = plsc.VectorSubcoreMesh(core_axis_name="c", subcore_axis_name="s")
    return pl.kernel(body, out_type=out_type, mesh=mesh,
                     scratch_types=scratch, name=f"sc_edge_d{D}",
                     compiler_params=pltpu.CompilerParams(
                         use_tc_tiling_on_sc=False))


def _tc_proj1(x_ref, wdT_ref, wsT_ref, ba_ref, p_ref, q_ref):
    x = x_ref[...]
    p_ref[...] = jnp.dot(x, wdT_ref[...], preferred_element_type=F32) + ba_ref[...]
    q_ref[...] = jnp.dot(x, wsT_ref[...], preferred_element_type=F32)


def _tc_mid(sp_ref, cp_ref, wb1T_ref, bb1_ref, wd2T_ref, ws2T_ref, ba2_ref,
            p2_ref, q2_ref):
    s = sp_ref[0] + sp_ref[1]
    cnt = cp_ref[0] + cp_ref[1]
    inv = 1.0 / jnp.maximum(cnt, 1.0)
    ind = (cnt > 0.5).astype(F32)
    agg = jnp.dot(s * inv, wb1T_ref[...], preferred_element_type=F32)
    h1 = jnp.maximum(agg + bb1_ref[...] * ind, 0.0)
    p2_ref[...] = jnp.dot(h1, wd2T_ref[...], preferred_element_type=F32) + ba2_ref[...]
    q2_ref[...] = jnp.dot(h1, ws2T_ref[...], preferred_element_type=F32)


def _tc_head(sp2_ref, cp_ref, wb2T_ref, bb2_ref, state_ref,
             fc1T_ref, fc1b_ref, fc2T_ref, fc2b_ref,
             meanT_ref, meanb_ref, lsT_ref, lsb_ref,
             mean_ref, ls_ref):
    s = sp2_ref[0] + sp2_ref[1]
    cnt = cp_ref[0] + cp_ref[1]
    inv = 1.0 / jnp.maximum(cnt, 1.0)
    ind = (cnt > 0.5).astype(F32)
    agg = jnp.dot(s * inv, wb2T_ref[...], preferred_element_type=F32)
    agg = agg + bb2_ref[...] * ind
    ge = jnp.sum(agg, axis=0, keepdims=True) * (1.0 / N_NODES)
    z = jnp.concatenate([state_ref[...], ge], axis=1)
    z = jnp.maximum(jnp.dot(z, fc1T_ref[...], preferred_element_type=F32)
                    + fc1b_ref[...], 0.0)
    z = jnp.maximum(jnp.dot(z, fc2T_ref[...], preferred_element_type=F32)
                    + fc2b_ref[...], 0.0)
    mean_ref[...] = jnp.dot(z, meanT_ref[...], preferred_element_type=F32) + meanb_ref[...]
    ls_ref[...] = jnp.clip(
        jnp.dot(z, lsT_ref[...], preferred_element_type=F32) + lsb_ref[...],
        -20.0, 2.0)


def kernel(state, x, edge_index, g1_Wa, g1_ba, g1_Wb, g1_bb,
           g2_Wa, g2_ba, g2_Wb, g2_bb, fc1_W, fc1_b, fc2_W, fc2_b,
           mean_W, mean_b, ls_W, ls_b):
    S = x.shape[1]
    H = g1_Wb.shape[0]
    G = g2_Wb.shape[0]
    src = edge_index[0].astype(jnp.int32)
    dst = edge_index[1].astype(jnp.int32)

    p1, q1 = pl.pallas_call(
        _tc_proj1,
        out_shape=[jax.ShapeDtypeStruct((N_NODES, H), F32),
                   jax.ShapeDtypeStruct((N_NODES, H), F32)],
    )(x, g1_Wa[:, :S].T, g1_Wa[:, S:].T, g1_ba.reshape(1, H))

    sp1, cp = _sc_edge_kernel(H, True)(p1, q1, dst, src)
    cp3 = cp.reshape(NC, N_PAD, 1)

    p2, q2 = pl.pallas_call(
        _tc_mid,
        out_shape=[jax.ShapeDtypeStruct((N_PAD, G), F32),
                   jax.ShapeDtypeStruct((N_PAD, G), F32)],
    )(sp1, cp3, g1_Wb.T, g1_bb.reshape(1, H), g2_Wa[:, :H].T,
      g2_Wa[:, H:].T, g2_ba.reshape(1, G))

    (sp2,) = _sc_edge_kernel(G, False)(p2, q2, dst, src)

    mean, log_std = pl.pallas_call(
        _tc_head,
        out_shape=[jax.ShapeDtypeStruct((1, mean_W.shape[0]), F32),
                   jax.ShapeDtypeStruct((1, mean_W.shape[0]), F32)],
    )(sp2, cp3, g2_Wb.T, g2_bb.reshape(1, G), state,
      fc1_W.T, fc1_b.reshape(1, -1), fc2_W.T, fc2_b.reshape(1, -1),
      mean_W.T, mean_b.reshape(1, -1), ls_W.T, ls_b.reshape(1, -1))
    return (mean, log_std)
